# baseline (speedup 1.0000x reference)
import jax
import jax.numpy as jnp
from jax import lax
from jax.experimental import pallas as pl
from jax.experimental.pallas import tpu as pltpu
from jax.experimental.pallas import tpu_sc as plsc

N = 16384
NNZ = 262144
D = 64
NC, NS = 2, 16
NW = NC * NS


def _mesh():
    return plsc.VectorSubcoreMesh(core_axis_name="c", subcore_axis_name="s",
                                  num_cores=NC, num_subcores=NS)


HALF = N // 2
TRASH = HALF
TROWS = 10240
SEG_ROWS_PER_TILE = NNZ // NS


def _seg_body(vals_hbm, idx_hbm, ramp_hbm, outr_hbm, outc_hbm,
              vals_v, idx_v, idxm_v, ramp_v, table_r, table_c):
    core = lax.axis_index("c")
    sub = lax.axis_index("s")

    def zrow(i, _):
        for k in range(D // 16):
            vals_v[i, pl.ds(k * 16, 16)] = jnp.zeros((16,), jnp.float32)
        return 0
    lax.fori_loop(0, 128, zrow, 0)
    pltpu.sync_copy(ramp_hbm.at[sub], ramp_v)

    def run(table, out_hbm, c):
        for p in range(2):
            for h in range(5):
                pltpu.sync_copy(vals_v.at[pl.ds(0, 128)],
                                table.at[ramp_v.at[h]])
            plsc.subcore_barrier()

            def chunk(g, _):
                sbase = pl.multiple_of(sub * SEG_ROWS_PER_TILE + g * 1024, 1024)
                pltpu.sync_copy(
                    idx_hbm.at[c, pl.ds(pl.multiple_of(sbase // 128, 8), 8)],
                    idx_v)
                lo = jnp.int32(p * HALF)
                for r in range(8):
                    for s in range(8):
                        sl = pl.ds(s * 16, 16)
                        iv = idx_v[r, sl]
                        loc = iv - lo
                        ok = (loc >= 0) & (loc < HALF)
                        idxm_v[r, sl] = jnp.where(ok, loc, jnp.int32(TRASH))
                for h in range(8):
                    base = pl.multiple_of(sbase + h * 128, 128)
                    pltpu.sync_copy(vals_hbm.at[pl.ds(base, 128)],
                                    vals_v.at[pl.ds(128, 128)])
                    pltpu.sync_copy(vals_v.at[pl.ds(128, 128)],
                                    table.at[idxm_v.at[h]], add=True)
                return 0
            lax.fori_loop(0, SEG_ROWS_PER_TILE // 1024, chunk, 0)
            plsc.subcore_barrier()

            for h in range(4):
                pltpu.sync_copy(table.at[ramp_v.at[8 + h]],
                                vals_v.at[pl.ds(128, 128)])
                ob = pl.multiple_of(p * HALF + sub * 512 + h * 128, 128)
                pltpu.sync_copy(vals_v.at[pl.ds(128, 128)],
                                out_hbm.at[pl.ds(ob, 128)])
            plsc.subcore_barrier()

    @pl.when(core == 0)
    def _():
        run(table_r, outr_hbm, 0)

    @pl.when(core == 1)
    def _():
        run(table_c, outc_hbm, 1)


def _seg_ramp():
    lane = jnp.arange(128, dtype=jnp.int32)
    sub_i = jnp.arange(NS, dtype=jnp.int32)
    slot = jnp.arange(16, dtype=jnp.int32)
    base = jnp.where(slot[None, :] < 8,
                     sub_i[:, None] * 640 + slot[None, :] * 128,
                     sub_i[:, None] * 512 + (slot[None, :] - 8) * 128)
    return (base[:, :, None] + lane[None, None, :]).astype(jnp.int32)


def _segment_sums(values, idx3, ramp):
    f = pl.kernel(
        _seg_body,
        out_type=[jax.ShapeDtypeStruct((N, D), jnp.float32),
                  jax.ShapeDtypeStruct((N, D), jnp.float32)],
        mesh=_mesh(),
        scratch_types=[
            pltpu.VMEM((128 + 128, D), jnp.float32),
            pltpu.VMEM((8, 128), jnp.int32),
            pltpu.VMEM((8, 128), jnp.int32),
            pltpu.VMEM((16, 128), jnp.int32),
            pltpu.VMEM_SHARED((TROWS, D), jnp.float32),
            pltpu.VMEM_SHARED((TROWS, D), jnp.float32),
        ],
    )
    return f(values, idx3, ramp)


MM_BLK = 2048


def _mm_body(v_ref, w_ref, o_ref):
    o_ref[...] = lax.dot_general(v_ref[...], w_ref[...],
                                 (((1,), (1,)), ((), ())),
                                 preferred_element_type=jnp.float32)


def _matmul_a(values, w1):
    return pl.pallas_call(
        _mm_body,
        grid=(NNZ // MM_BLK,),
        in_specs=[pl.BlockSpec((MM_BLK, D), lambda i: (i, 0)),
                  pl.BlockSpec((D, D), lambda i: (0, 0))],
        out_specs=pl.BlockSpec((MM_BLK, D), lambda i: (i, 0)),
        out_shape=jax.ShapeDtypeStruct((NNZ, D), jnp.float32),
    )(values, w1)


def _small_body(rsum_ref, csum_ref, w_ref, b_ref, t_ref, c_ref):
    row_sum = rsum_ref[...]
    col_sum = csum_ref[...]
    w = w_ref[...]
    w2 = w[:, D:2 * D]
    w3 = w[:, 2 * D:3 * D]
    w4 = w[:, 3 * D:4 * D]
    dims = (((1,), (1,)), ((), ()))
    trow = lax.dot_general(row_sum, w3, dims, preferred_element_type=jnp.float32)
    tcol = lax.dot_general(col_sum, w2, dims, preferred_element_type=jnp.float32)
    t_ref[...] = jnp.concatenate([trow, tcol], axis=1)
    total = jnp.sum(row_sum, axis=0, keepdims=True) * jnp.float32(1.0 / NNZ)
    c_ref[...] = lax.dot_general(total, w4, dims,
                                 preferred_element_type=jnp.float32) + b_ref[...]


def _small(rsum, csum, w, b2d):
    return pl.pallas_call(
        _small_body,
        out_shape=[jax.ShapeDtypeStruct((N, 2 * D), jnp.float32),
                   jax.ShapeDtypeStruct((1, D), jnp.float32)],
    )(rsum, csum, w, b2d)


C2 = 256
RW = NNZ // NW
C2_ITERS = RW // C2


def _comb_body(a_hbm, t_hbm, rowf_hbm, colf_hbm, c_hbm, out_hbm,
               a_v, gr_v, gc_v, idxr_v, idxc_v, c_v, sem):
    core = lax.axis_index("c")
    sub = lax.axis_index("s")
    wid = sub * NC + core

    pltpu.sync_copy(c_hbm, c_v)
    cs = [c_v[k] for k in range(D // 16)]

    def chunk(g, _):
        base = pl.multiple_of(wid * RW + g * C2, C2)
        pltpu.sync_copy(a_hbm.at[pl.ds(base, C2)], a_v)
        pltpu.sync_copy(rowf_hbm.at[pl.ds(base, C2)], idxr_v)
        pltpu.sync_copy(colf_hbm.at[pl.ds(base, C2)], idxc_v)
        cps = []
        for j in range(C2 // 128):
            cps.append(pltpu.async_copy(t_hbm.at[idxr_v.at[pl.ds(j * 128, 128)]],
                                        gr_v.at[pl.ds(j * 128, 128)], sem))
            cps.append(pltpu.async_copy(t_hbm.at[idxc_v.at[pl.ds(j * 128, 128)]],
                                        gc_v.at[pl.ds(j * 128, 128)], sem))
        for cp in cps:
            cp.wait()

        def rowfn(i, _):
            for k in range(D // 16):
                sl = pl.ds(k * 16, 16)
                t = (a_v[i, sl] + gr_v[i, sl]
                     + gc_v[i, pl.ds(D + k * 16, 16)] + cs[k])
                a_v[i, sl] = jnp.maximum(t, t * jnp.float32(0.01))
            return 0
        lax.fori_loop(0, C2, rowfn, 0)
        pltpu.sync_copy(a_v, out_hbm.at[pl.ds(base, C2)])
        return 0
    lax.fori_loop(0, C2_ITERS, chunk, 0)


def _combine(a, t, rowf, colf, c4):
    f = pl.kernel(
        _comb_body,
        out_type=jax.ShapeDtypeStruct((NNZ, D), jnp.float32),
        mesh=_mesh(),
        scratch_types=[
            pltpu.VMEM((C2, D), jnp.float32),
            pltpu.VMEM((C2, 2 * D), jnp.float32),
            pltpu.VMEM((C2, 2 * D), jnp.float32),
            pltpu.VMEM((C2,), jnp.int32),
            pltpu.VMEM((C2,), jnp.int32),
            pltpu.VMEM((D // 16, 16), jnp.float32),
            pltpu.SemaphoreType.DMA,
        ],
    )
    return f(a, t, rowf, colf, c4)


TSEG_BLK = 8192


def _tcseg_body(idx_ref, vals_ref, out_ref, acc):
    g = pl.program_id(0)

    @pl.when(g == 0)
    def _():
        acc[...] = jnp.zeros_like(acc)

    def body(i, _):
        r = idx_ref[i]
        acc[pl.ds(r, 1), :] += vals_ref[pl.ds(i, 1), :]
        return 0
    lax.fori_loop(0, TSEG_BLK, body, 0)

    @pl.when(g == (NNZ // TSEG_BLK) - 1)
    def _():
        out_ref[...] = acc[...]


def _tc_segment_sum(values, idx):
    return pl.pallas_call(
        _tcseg_body,
        grid=(NNZ // TSEG_BLK,),
        in_specs=[pl.BlockSpec((TSEG_BLK,), lambda i: (i,),
                               memory_space=pltpu.SMEM),
                  pl.BlockSpec((TSEG_BLK, D), lambda i: (i, 0))],
        out_specs=pl.BlockSpec((N, D), lambda i: (0, 0)),
        out_shape=jax.ShapeDtypeStruct((N, D), jnp.float32),
        scratch_shapes=[pltpu.VMEM((N, D), jnp.float32)],
    )(idx, values)


def kernel(values, indices, W, b):
    row_sum = _tc_segment_sum(values, indices[0])
    col_sum = _tc_segment_sum(values, indices[1])
    a = _matmul_a(values, W[:, :D])
    t, c = _small(row_sum, col_sum, W, b.reshape(1, D))
    return _combine(a, t, indices[0], indices[1], c.reshape(D // 16, 16))


# speedup vs baseline: 1.6064x; 1.6064x over previous
import jax
import jax.numpy as jnp
from jax import lax
from jax.experimental import pallas as pl
from jax.experimental.pallas import tpu as pltpu
from jax.experimental.pallas import tpu_sc as plsc

N = 16384
NNZ = 262144
D = 64
NC, NS = 2, 16
NW = NC * NS


def _mesh():
    return plsc.VectorSubcoreMesh(core_axis_name="c", subcore_axis_name="s",
                                  num_cores=NC, num_subcores=NS)


HALF = N // 2
TRASH = HALF
TROWS = 10240
SEG_ROWS_PER_TILE = NNZ // NS


def _seg_body(vals_hbm, idx_hbm, ramp_hbm, outr_hbm, outc_hbm,
              vals_v, idx_v, idxm_v, ramp_v, table_r, table_c):
    core = lax.axis_index("c")
    sub = lax.axis_index("s")

    def zrow(i, _):
        for k in range(D // 16):
            vals_v[i, pl.ds(k * 16, 16)] = jnp.zeros((16,), jnp.float32)
        return 0
    lax.fori_loop(0, 128, zrow, 0)
    pltpu.sync_copy(ramp_hbm.at[sub], ramp_v)

    def run(table, out_hbm, c):
        for p in range(2):
            for h in range(5):
                pltpu.sync_copy(vals_v.at[pl.ds(0, 128)],
                                table.at[ramp_v.at[h]])
            plsc.subcore_barrier()

            def chunk(g, _):
                sbase = pl.multiple_of(sub * SEG_ROWS_PER_TILE + g * 1024, 1024)
                pltpu.sync_copy(
                    idx_hbm.at[c, pl.ds(pl.multiple_of(sbase // 128, 8), 8)],
                    idx_v)
                lo = jnp.int32(p * HALF)
                for r in range(8):
                    for s in range(8):
                        sl = pl.ds(s * 16, 16)
                        iv = idx_v[r, sl]
                        loc = iv - lo
                        ok = (loc >= 0) & (loc < HALF)
                        idxm_v[r, sl] = jnp.where(ok, loc, jnp.int32(TRASH))
                for h in range(8):
                    base = pl.multiple_of(sbase + h * 128, 128)
                    pltpu.sync_copy(vals_hbm.at[pl.ds(base, 128)],
                                    vals_v.at[pl.ds(128, 128)])
                    pltpu.sync_copy(vals_v.at[pl.ds(128, 128)],
                                    table.at[idxm_v.at[h]], add=True)
                return 0
            lax.fori_loop(0, SEG_ROWS_PER_TILE // 1024, chunk, 0)
            plsc.subcore_barrier()

            for h in range(4):
                pltpu.sync_copy(table.at[ramp_v.at[8 + h]],
                                vals_v.at[pl.ds(128, 128)])
                ob = pl.multiple_of(p * HALF + sub * 512 + h * 128, 128)
                pltpu.sync_copy(vals_v.at[pl.ds(128, 128)],
                                out_hbm.at[pl.ds(ob, 128)])
            plsc.subcore_barrier()

    @pl.when(core == 0)
    def _():
        run(table_r, outr_hbm, 0)

    @pl.when(core == 1)
    def _():
        run(table_c, outc_hbm, 1)


def _seg_ramp():
    lane = jnp.arange(128, dtype=jnp.int32)
    sub_i = jnp.arange(NS, dtype=jnp.int32)
    slot = jnp.arange(16, dtype=jnp.int32)
    base = jnp.where(slot[None, :] < 8,
                     sub_i[:, None] * 640 + slot[None, :] * 128,
                     sub_i[:, None] * 512 + (slot[None, :] - 8) * 128)
    return (base[:, :, None] + lane[None, None, :]).astype(jnp.int32)


def _segment_sums(values, idx3, ramp):
    f = pl.kernel(
        _seg_body,
        out_type=[jax.ShapeDtypeStruct((N, D), jnp.float32),
                  jax.ShapeDtypeStruct((N, D), jnp.float32)],
        mesh=_mesh(),
        scratch_types=[
            pltpu.VMEM((128 + 128, D), jnp.float32),
            pltpu.VMEM((8, 128), jnp.int32),
            pltpu.VMEM((8, 128), jnp.int32),
            pltpu.VMEM((16, 128), jnp.int32),
            pltpu.VMEM_SHARED((TROWS, D), jnp.float32),
            pltpu.VMEM_SHARED((TROWS, D), jnp.float32),
        ],
    )
    return f(values, idx3, ramp)


MM_BLK = 2048


def _mm_body(v_ref, w_ref, o_ref):
    o_ref[...] = lax.dot_general(v_ref[...], w_ref[...],
                                 (((1,), (1,)), ((), ())),
                                 preferred_element_type=jnp.float32)


def _matmul_a(values, w1):
    return pl.pallas_call(
        _mm_body,
        grid=(NNZ // MM_BLK,),
        in_specs=[pl.BlockSpec((MM_BLK, D), lambda i: (i, 0)),
                  pl.BlockSpec((D, D), lambda i: (0, 0))],
        out_specs=pl.BlockSpec((MM_BLK, D), lambda i: (i, 0)),
        out_shape=jax.ShapeDtypeStruct((NNZ, D), jnp.float32),
    )(values, w1)


def _small_body(rsum_ref, csum_ref, w_ref, b_ref, t_ref, c_ref):
    row_sum = rsum_ref[...]
    col_sum = csum_ref[...]
    w = w_ref[...]
    w2 = w[:, D:2 * D]
    w3 = w[:, 2 * D:3 * D]
    w4 = w[:, 3 * D:4 * D]
    dims = (((1,), (1,)), ((), ()))
    trow = lax.dot_general(row_sum, w3, dims, preferred_element_type=jnp.float32)
    tcol = lax.dot_general(col_sum, w2, dims, preferred_element_type=jnp.float32)
    t_ref[...] = jnp.concatenate([trow, tcol], axis=1)
    total = jnp.sum(row_sum, axis=0, keepdims=True) * jnp.float32(1.0 / NNZ)
    c_ref[...] = lax.dot_general(total, w4, dims,
                                 preferred_element_type=jnp.float32) + b_ref[...]


def _small(rsum, csum, w, b2d):
    return pl.pallas_call(
        _small_body,
        out_shape=[jax.ShapeDtypeStruct((N, 2 * D), jnp.float32),
                   jax.ShapeDtypeStruct((1, D), jnp.float32)],
    )(rsum, csum, w, b2d)


C2 = 256
RW = NNZ // NW
C2_ITERS = RW // C2


def _comb_body(a_hbm, t_hbm, rowf_hbm, colf_hbm, c_hbm, out_hbm,
               a_v, gr_v, gc_v, idxr_v, idxc_v, c_v, sem):
    core = lax.axis_index("c")
    sub = lax.axis_index("s")
    wid = sub * NC + core

    pltpu.sync_copy(c_hbm, c_v)
    cs = [c_v[k] for k in range(D // 16)]

    def chunk(g, _):
        base = pl.multiple_of(wid * RW + g * C2, C2)
        pltpu.sync_copy(a_hbm.at[pl.ds(base, C2)], a_v)
        pltpu.sync_copy(rowf_hbm.at[pl.ds(base, C2)], idxr_v)
        pltpu.sync_copy(colf_hbm.at[pl.ds(base, C2)], idxc_v)
        cps = []
        for j in range(C2 // 128):
            cps.append(pltpu.async_copy(t_hbm.at[idxr_v.at[pl.ds(j * 128, 128)]],
                                        gr_v.at[pl.ds(j * 128, 128)], sem))
            cps.append(pltpu.async_copy(t_hbm.at[idxc_v.at[pl.ds(j * 128, 128)]],
                                        gc_v.at[pl.ds(j * 128, 128)], sem))
        for cp in cps:
            cp.wait()

        def rowfn(i, _):
            for k in range(D // 16):
                sl = pl.ds(k * 16, 16)
                t = (a_v[i, sl] + gr_v[i, sl]
                     + gc_v[i, pl.ds(D + k * 16, 16)] + cs[k])
                a_v[i, sl] = jnp.maximum(t, t * jnp.float32(0.01))
            return 0
        lax.fori_loop(0, C2, rowfn, 0)
        pltpu.sync_copy(a_v, out_hbm.at[pl.ds(base, C2)])
        return 0
    lax.fori_loop(0, C2_ITERS, chunk, 0)


def _combine(a, t, rowf, colf, c4):
    f = pl.kernel(
        _comb_body,
        out_type=jax.ShapeDtypeStruct((NNZ, D), jnp.float32),
        mesh=_mesh(),
        scratch_types=[
            pltpu.VMEM((C2, D), jnp.float32),
            pltpu.VMEM((C2, 2 * D), jnp.float32),
            pltpu.VMEM((C2, 2 * D), jnp.float32),
            pltpu.VMEM((C2,), jnp.int32),
            pltpu.VMEM((C2,), jnp.int32),
            pltpu.VMEM((D // 16, 16), jnp.float32),
            pltpu.SemaphoreType.DMA,
        ],
    )
    return f(a, t, rowf, colf, c4)


TSEG_BLK = 8192
NACC = 2


def _tcseg_body(idx_ref, vals_ref, out_ref, *accs):
    g = pl.program_id(0)

    @pl.when(g == 0)
    def _():
        for a in accs:
            a[...] = jnp.zeros_like(a)

    def body(i, _):
        for k in range(NACC):
            r = idx_ref[i * NACC + k]
            accs[k][pl.ds(r, 1), :] += vals_ref[pl.ds(i * NACC + k, 1), :]
        return 0
    lax.fori_loop(0, TSEG_BLK // NACC, body, 0)

    @pl.when(g == (NNZ // TSEG_BLK) - 1)
    def _():
        s = accs[0][...]
        for a in accs[1:]:
            s = s + a[...]
        out_ref[...] = s


def _tc_segment_sum(values, idx):
    return pl.pallas_call(
        _tcseg_body,
        grid=(NNZ // TSEG_BLK,),
        in_specs=[pl.BlockSpec((TSEG_BLK,), lambda i: (i,),
                               memory_space=pltpu.SMEM),
                  pl.BlockSpec((TSEG_BLK, D), lambda i: (i, 0))],
        out_specs=pl.BlockSpec((N, D), lambda i: (0, 0)),
        out_shape=jax.ShapeDtypeStruct((N, D), jnp.float32),
        scratch_shapes=[pltpu.VMEM((N, D), jnp.float32)] * NACC,
    )(idx, values)


def kernel(values, indices, W, b):
    row_sum = _tc_segment_sum(values, indices[0])
    col_sum = _tc_segment_sum(values, indices[1])
    a = _matmul_a(values, W[:, :D])
    t, c = _small(row_sum, col_sum, W, b.reshape(1, D))
    return _combine(a, t, indices[0], indices[1], c.reshape(D // 16, 16))


# speedup vs baseline: 2.1380x; 1.3310x over previous
import jax
import jax.numpy as jnp
from jax import lax
from jax.experimental import pallas as pl
from jax.experimental.pallas import tpu as pltpu
from jax.experimental.pallas import tpu_sc as plsc

N = 16384
NNZ = 262144
D = 64
NC, NS = 2, 16
NW = NC * NS


def _mesh():
    return plsc.VectorSubcoreMesh(core_axis_name="c", subcore_axis_name="s",
                                  num_cores=NC, num_subcores=NS)


HALF = N // 2
TRASH = HALF
TROWS = 10240
SEG_ROWS_PER_TILE = NNZ // NS


def _seg_body(vals_hbm, idx_hbm, ramp_hbm, outr_hbm, outc_hbm,
              vals_v, idx_v, idxm_v, ramp_v, table_r, table_c):
    core = lax.axis_index("c")
    sub = lax.axis_index("s")

    def zrow(i, _):
        for k in range(D // 16):
            vals_v[i, pl.ds(k * 16, 16)] = jnp.zeros((16,), jnp.float32)
        return 0
    lax.fori_loop(0, 128, zrow, 0)
    pltpu.sync_copy(ramp_hbm.at[sub], ramp_v)

    def run(table, out_hbm, c):
        for p in range(2):
            for h in range(5):
                pltpu.sync_copy(vals_v.at[pl.ds(0, 128)],
                                table.at[ramp_v.at[h]])
            plsc.subcore_barrier()

            def chunk(g, _):
                sbase = pl.multiple_of(sub * SEG_ROWS_PER_TILE + g * 1024, 1024)
                pltpu.sync_copy(
                    idx_hbm.at[c, pl.ds(pl.multiple_of(sbase // 128, 8), 8)],
                    idx_v)
                lo = jnp.int32(p * HALF)
                for r in range(8):
                    for s in range(8):
                        sl = pl.ds(s * 16, 16)
                        iv = idx_v[r, sl]
                        loc = iv - lo
                        ok = (loc >= 0) & (loc < HALF)
                        idxm_v[r, sl] = jnp.where(ok, loc, jnp.int32(TRASH))
                for h in range(8):
                    base = pl.multiple_of(sbase + h * 128, 128)
                    pltpu.sync_copy(vals_hbm.at[pl.ds(base, 128)],
                                    vals_v.at[pl.ds(128, 128)])
                    pltpu.sync_copy(vals_v.at[pl.ds(128, 128)],
                                    table.at[idxm_v.at[h]], add=True)
                return 0
            lax.fori_loop(0, SEG_ROWS_PER_TILE // 1024, chunk, 0)
            plsc.subcore_barrier()

            for h in range(4):
                pltpu.sync_copy(table.at[ramp_v.at[8 + h]],
                                vals_v.at[pl.ds(128, 128)])
                ob = pl.multiple_of(p * HALF + sub * 512 + h * 128, 128)
                pltpu.sync_copy(vals_v.at[pl.ds(128, 128)],
                                out_hbm.at[pl.ds(ob, 128)])
            plsc.subcore_barrier()

    @pl.when(core == 0)
    def _():
        run(table_r, outr_hbm, 0)

    @pl.when(core == 1)
    def _():
        run(table_c, outc_hbm, 1)


def _seg_ramp():
    lane = jnp.arange(128, dtype=jnp.int32)
    sub_i = jnp.arange(NS, dtype=jnp.int32)
    slot = jnp.arange(16, dtype=jnp.int32)
    base = jnp.where(slot[None, :] < 8,
                     sub_i[:, None] * 640 + slot[None, :] * 128,
                     sub_i[:, None] * 512 + (slot[None, :] - 8) * 128)
    return (base[:, :, None] + lane[None, None, :]).astype(jnp.int32)


def _segment_sums(values, idx3, ramp):
    f = pl.kernel(
        _seg_body,
        out_type=[jax.ShapeDtypeStruct((N, D), jnp.float32),
                  jax.ShapeDtypeStruct((N, D), jnp.float32)],
        mesh=_mesh(),
        scratch_types=[
            pltpu.VMEM((128 + 128, D), jnp.float32),
            pltpu.VMEM((8, 128), jnp.int32),
            pltpu.VMEM((8, 128), jnp.int32),
            pltpu.VMEM((16, 128), jnp.int32),
            pltpu.VMEM_SHARED((TROWS, D), jnp.float32),
            pltpu.VMEM_SHARED((TROWS, D), jnp.float32),
        ],
    )
    return f(values, idx3, ramp)


MM_BLK = 2048


def _mm_body(v_ref, w_ref, o_ref):
    o_ref[...] = lax.dot_general(v_ref[...], w_ref[...],
                                 (((1,), (1,)), ((), ())),
                                 preferred_element_type=jnp.float32)


def _matmul_a(values, w1):
    return pl.pallas_call(
        _mm_body,
        grid=(NNZ // MM_BLK,),
        in_specs=[pl.BlockSpec((MM_BLK, D), lambda i: (i, 0)),
                  pl.BlockSpec((D, D), lambda i: (0, 0))],
        out_specs=pl.BlockSpec((MM_BLK, D), lambda i: (i, 0)),
        out_shape=jax.ShapeDtypeStruct((NNZ, D), jnp.float32),
    )(values, w1)


def _small_body(rsum_ref, csum_ref, w_ref, b_ref, t_ref, c_ref):
    row_sum = rsum_ref[...]
    col_sum = csum_ref[...]
    w = w_ref[...]
    w2 = w[:, D:2 * D]
    w3 = w[:, 2 * D:3 * D]
    w4 = w[:, 3 * D:4 * D]
    dims = (((1,), (1,)), ((), ()))
    trow = lax.dot_general(row_sum, w3, dims, preferred_element_type=jnp.float32)
    tcol = lax.dot_general(col_sum, w2, dims, preferred_element_type=jnp.float32)
    t_ref[...] = jnp.concatenate([trow, tcol], axis=1)
    total = jnp.sum(row_sum, axis=0, keepdims=True) * jnp.float32(1.0 / NNZ)
    c_ref[...] = lax.dot_general(total, w4, dims,
                                 preferred_element_type=jnp.float32) + b_ref[...]


def _small(rsum, csum, w, b2d):
    return pl.pallas_call(
        _small_body,
        out_shape=[jax.ShapeDtypeStruct((N, 2 * D), jnp.float32),
                   jax.ShapeDtypeStruct((1, D), jnp.float32)],
    )(rsum, csum, w, b2d)


C2 = 256
RW = NNZ // NW
C2_ITERS = RW // C2


def _comb_body(a_hbm, t_hbm, rowf_hbm, colf_hbm, c_hbm, out_hbm,
               a_v, gr_v, gc_v, idxr_v, idxc_v, c_v, sem):
    core = lax.axis_index("c")
    sub = lax.axis_index("s")
    wid = sub * NC + core

    pltpu.sync_copy(c_hbm, c_v)
    cs = [c_v[k] for k in range(D // 16)]

    def chunk(g, _):
        base = pl.multiple_of(wid * RW + g * C2, C2)
        pltpu.sync_copy(a_hbm.at[pl.ds(base, C2)], a_v)
        pltpu.sync_copy(rowf_hbm.at[pl.ds(base, C2)], idxr_v)
        pltpu.sync_copy(colf_hbm.at[pl.ds(base, C2)], idxc_v)
        cps = []
        for j in range(C2 // 128):
            cps.append(pltpu.async_copy(t_hbm.at[idxr_v.at[pl.ds(j * 128, 128)]],
                                        gr_v.at[pl.ds(j * 128, 128)], sem))
            cps.append(pltpu.async_copy(t_hbm.at[idxc_v.at[pl.ds(j * 128, 128)]],
                                        gc_v.at[pl.ds(j * 128, 128)], sem))
        for cp in cps:
            cp.wait()

        def rowfn(i, _):
            for k in range(D // 16):
                sl = pl.ds(k * 16, 16)
                t = (a_v[i, sl] + gr_v[i, sl]
                     + gc_v[i, pl.ds(D + k * 16, 16)] + cs[k])
                a_v[i, sl] = jnp.maximum(t, t * jnp.float32(0.01))
            return 0
        lax.fori_loop(0, C2, rowfn, 0)
        pltpu.sync_copy(a_v, out_hbm.at[pl.ds(base, C2)])
        return 0
    lax.fori_loop(0, C2_ITERS, chunk, 0)


def _combine(a, t, rowf, colf, c4):
    f = pl.kernel(
        _comb_body,
        out_type=jax.ShapeDtypeStruct((NNZ, D), jnp.float32),
        mesh=_mesh(),
        scratch_types=[
            pltpu.VMEM((C2, D), jnp.float32),
            pltpu.VMEM((C2, 2 * D), jnp.float32),
            pltpu.VMEM((C2, 2 * D), jnp.float32),
            pltpu.VMEM((C2,), jnp.int32),
            pltpu.VMEM((C2,), jnp.int32),
            pltpu.VMEM((D // 16, 16), jnp.float32),
            pltpu.SemaphoreType.DMA,
        ],
    )
    return f(a, t, rowf, colf, c4)


TSEG_BLK = 4096
NACC = 4


def _tcseg_body(idx_ref, vals_ref, out_ref, *accs):
    g = pl.program_id(0)

    @pl.when(g == 0)
    def _():
        for a in accs:
            a[...] = jnp.zeros_like(a)

    def body(i, _):
        for k in range(NACC):
            r = idx_ref[i * NACC + k]
            accs[k][pl.ds(r, 1), :] += vals_ref[pl.ds(i * NACC + k, 1), :]
        return 0
    lax.fori_loop(0, TSEG_BLK // NACC, body, 0)

    @pl.when(g == (NNZ // TSEG_BLK) - 1)
    def _():
        s = accs[0][...]
        for a in accs[1:]:
            s = s + a[...]
        out_ref[...] = s


def _tc_segment_sum(values, idx):
    return pl.pallas_call(
        _tcseg_body,
        grid=(NNZ // TSEG_BLK,),
        in_specs=[pl.BlockSpec((TSEG_BLK,), lambda i: (i,),
                               memory_space=pltpu.SMEM),
                  pl.BlockSpec((TSEG_BLK, D), lambda i: (i, 0))],
        out_specs=pl.BlockSpec((N, D), lambda i: (0, 0)),
        out_shape=jax.ShapeDtypeStruct((N, D), jnp.float32),
        scratch_shapes=[pltpu.VMEM((N, D), jnp.float32)] * NACC,
    )(idx, values)


def kernel(values, indices, W, b):
    row_sum = _tc_segment_sum(values, indices[0])
    col_sum = _tc_segment_sum(values, indices[1])
    a = _matmul_a(values, W[:, :D])
    t, c = _small(row_sum, col_sum, W, b.reshape(1, D))
    return _combine(a, t, indices[0], indices[1], c.reshape(D // 16, 16))


# speedup vs baseline: 2.2244x; 1.0404x over previous
import jax
import jax.numpy as jnp
from jax import lax
from jax.experimental import pallas as pl
from jax.experimental.pallas import tpu as pltpu
from jax.experimental.pallas import tpu_sc as plsc

N = 16384
NNZ = 262144
D = 64
NC, NS = 2, 16
NW = NC * NS


def _mesh():
    return plsc.VectorSubcoreMesh(core_axis_name="c", subcore_axis_name="s",
                                  num_cores=NC, num_subcores=NS)


HALF = N // 2
TRASH = HALF
TROWS = 10240
SEG_ROWS_PER_TILE = NNZ // NS


def _seg_body(vals_hbm, idx_hbm, ramp_hbm, outr_hbm, outc_hbm,
              vals_v, idx_v, idxm_v, ramp_v, table_r, table_c):
    core = lax.axis_index("c")
    sub = lax.axis_index("s")

    def zrow(i, _):
        for k in range(D // 16):
            vals_v[i, pl.ds(k * 16, 16)] = jnp.zeros((16,), jnp.float32)
        return 0
    lax.fori_loop(0, 128, zrow, 0)
    pltpu.sync_copy(ramp_hbm.at[sub], ramp_v)

    def run(table, out_hbm, c):
        for p in range(2):
            for h in range(5):
                pltpu.sync_copy(vals_v.at[pl.ds(0, 128)],
                                table.at[ramp_v.at[h]])
            plsc.subcore_barrier()

            def chunk(g, _):
                sbase = pl.multiple_of(sub * SEG_ROWS_PER_TILE + g * 1024, 1024)
                pltpu.sync_copy(
                    idx_hbm.at[c, pl.ds(pl.multiple_of(sbase // 128, 8), 8)],
                    idx_v)
                lo = jnp.int32(p * HALF)
                for r in range(8):
                    for s in range(8):
                        sl = pl.ds(s * 16, 16)
                        iv = idx_v[r, sl]
                        loc = iv - lo
                        ok = (loc >= 0) & (loc < HALF)
                        idxm_v[r, sl] = jnp.where(ok, loc, jnp.int32(TRASH))
                for h in range(8):
                    base = pl.multiple_of(sbase + h * 128, 128)
                    pltpu.sync_copy(vals_hbm.at[pl.ds(base, 128)],
                                    vals_v.at[pl.ds(128, 128)])
                    pltpu.sync_copy(vals_v.at[pl.ds(128, 128)],
                                    table.at[idxm_v.at[h]], add=True)
                return 0
            lax.fori_loop(0, SEG_ROWS_PER_TILE // 1024, chunk, 0)
            plsc.subcore_barrier()

            for h in range(4):
                pltpu.sync_copy(table.at[ramp_v.at[8 + h]],
                                vals_v.at[pl.ds(128, 128)])
                ob = pl.multiple_of(p * HALF + sub * 512 + h * 128, 128)
                pltpu.sync_copy(vals_v.at[pl.ds(128, 128)],
                                out_hbm.at[pl.ds(ob, 128)])
            plsc.subcore_barrier()

    @pl.when(core == 0)
    def _():
        run(table_r, outr_hbm, 0)

    @pl.when(core == 1)
    def _():
        run(table_c, outc_hbm, 1)


def _seg_ramp():
    lane = jnp.arange(128, dtype=jnp.int32)
    sub_i = jnp.arange(NS, dtype=jnp.int32)
    slot = jnp.arange(16, dtype=jnp.int32)
    base = jnp.where(slot[None, :] < 8,
                     sub_i[:, None] * 640 + slot[None, :] * 128,
                     sub_i[:, None] * 512 + (slot[None, :] - 8) * 128)
    return (base[:, :, None] + lane[None, None, :]).astype(jnp.int32)


def _segment_sums(values, idx3, ramp):
    f = pl.kernel(
        _seg_body,
        out_type=[jax.ShapeDtypeStruct((N, D), jnp.float32),
                  jax.ShapeDtypeStruct((N, D), jnp.float32)],
        mesh=_mesh(),
        scratch_types=[
            pltpu.VMEM((128 + 128, D), jnp.float32),
            pltpu.VMEM((8, 128), jnp.int32),
            pltpu.VMEM((8, 128), jnp.int32),
            pltpu.VMEM((16, 128), jnp.int32),
            pltpu.VMEM_SHARED((TROWS, D), jnp.float32),
            pltpu.VMEM_SHARED((TROWS, D), jnp.float32),
        ],
    )
    return f(values, idx3, ramp)


MM_BLK = 2048


def _mm_body(v_ref, w_ref, o_ref):
    o_ref[...] = lax.dot_general(v_ref[...], w_ref[...],
                                 (((1,), (1,)), ((), ())),
                                 preferred_element_type=jnp.float32)


def _matmul_a(values, w1):
    return pl.pallas_call(
        _mm_body,
        grid=(NNZ // MM_BLK,),
        in_specs=[pl.BlockSpec((MM_BLK, D), lambda i: (i, 0)),
                  pl.BlockSpec((D, D), lambda i: (0, 0))],
        out_specs=pl.BlockSpec((MM_BLK, D), lambda i: (i, 0)),
        out_shape=jax.ShapeDtypeStruct((NNZ, D), jnp.float32),
    )(values, w1)


def _small_body(rsum_ref, csum_ref, w_ref, b_ref, t_ref, c_ref):
    row_sum = rsum_ref[...]
    col_sum = csum_ref[...]
    w = w_ref[...]
    w2 = w[:, D:2 * D]
    w3 = w[:, 2 * D:3 * D]
    w4 = w[:, 3 * D:4 * D]
    dims = (((1,), (1,)), ((), ()))
    trow = lax.dot_general(row_sum, w3, dims, preferred_element_type=jnp.float32)
    tcol = lax.dot_general(col_sum, w2, dims, preferred_element_type=jnp.float32)
    t_ref[...] = jnp.concatenate([trow, tcol], axis=1)
    total = jnp.sum(row_sum, axis=0, keepdims=True) * jnp.float32(1.0 / NNZ)
    c_ref[...] = lax.dot_general(total, w4, dims,
                                 preferred_element_type=jnp.float32) + b_ref[...]


def _small(rsum, csum, w, b2d):
    return pl.pallas_call(
        _small_body,
        out_shape=[jax.ShapeDtypeStruct((N, 2 * D), jnp.float32),
                   jax.ShapeDtypeStruct((1, D), jnp.float32)],
    )(rsum, csum, w, b2d)


C2 = 256
RW = NNZ // NW
C2_ITERS = RW // C2


def _comb_body(a_hbm, t_hbm, rowf_hbm, colf_hbm, c_hbm, out_hbm,
               a_v, gr_v, gc_v, idxr_v, idxc_v, c_v, sem):
    core = lax.axis_index("c")
    sub = lax.axis_index("s")
    wid = sub * NC + core

    pltpu.sync_copy(c_hbm, c_v)
    cs = [c_v[k] for k in range(D // 16)]

    def chunk(g, _):
        base = pl.multiple_of(wid * RW + g * C2, C2)
        pltpu.sync_copy(a_hbm.at[pl.ds(base, C2)], a_v)
        pltpu.sync_copy(rowf_hbm.at[pl.ds(base, C2)], idxr_v)
        pltpu.sync_copy(colf_hbm.at[pl.ds(base, C2)], idxc_v)
        cps = []
        for j in range(C2 // 128):
            cps.append(pltpu.async_copy(t_hbm.at[idxr_v.at[pl.ds(j * 128, 128)]],
                                        gr_v.at[pl.ds(j * 128, 128)], sem))
            cps.append(pltpu.async_copy(t_hbm.at[idxc_v.at[pl.ds(j * 128, 128)]],
                                        gc_v.at[pl.ds(j * 128, 128)], sem))
        for cp in cps:
            cp.wait()

        def rowfn(i, _):
            for k in range(D // 16):
                sl = pl.ds(k * 16, 16)
                t = (a_v[i, sl] + gr_v[i, sl]
                     + gc_v[i, pl.ds(D + k * 16, 16)] + cs[k])
                a_v[i, sl] = jnp.maximum(t, t * jnp.float32(0.01))
            return 0
        lax.fori_loop(0, C2, rowfn, 0)
        pltpu.sync_copy(a_v, out_hbm.at[pl.ds(base, C2)])
        return 0
    lax.fori_loop(0, C2_ITERS, chunk, 0)


def _combine(a, t, rowf, colf, c4):
    f = pl.kernel(
        _comb_body,
        out_type=jax.ShapeDtypeStruct((NNZ, D), jnp.float32),
        mesh=_mesh(),
        scratch_types=[
            pltpu.VMEM((C2, D), jnp.float32),
            pltpu.VMEM((C2, 2 * D), jnp.float32),
            pltpu.VMEM((C2, 2 * D), jnp.float32),
            pltpu.VMEM((C2,), jnp.int32),
            pltpu.VMEM((C2,), jnp.int32),
            pltpu.VMEM((D // 16, 16), jnp.float32),
            pltpu.SemaphoreType.DMA,
        ],
    )
    return f(a, t, rowf, colf, c4)


TSEG_BLK = 4096


def _tcseg_body(ridx_ref, cidx_ref, vals_ref, outr_ref, outc_ref,
                ar0, ar1, ac0, ac1):
    g = pl.program_id(0)
    accs = (ar0, ar1, ac0, ac1)

    @pl.when(g == 0)
    def _():
        for a in accs:
            a[...] = jnp.zeros_like(a)

    def body(i, _):
        for k in range(2):
            v = vals_ref[pl.ds(i * 2 + k, 1), :]
            rr = ridx_ref[i * 2 + k]
            cc = cidx_ref[i * 2 + k]
            accs[k][pl.ds(rr, 1), :] += v
            accs[2 + k][pl.ds(cc, 1), :] += v
        return 0
    lax.fori_loop(0, TSEG_BLK // 2, body, 0)

    @pl.when(g == (NNZ // TSEG_BLK) - 1)
    def _():
        outr_ref[...] = ar0[...] + ar1[...]
        outc_ref[...] = ac0[...] + ac1[...]


def _tc_segment_sums(values, ridx, cidx):
    return pl.pallas_call(
        _tcseg_body,
        grid=(NNZ // TSEG_BLK,),
        in_specs=[pl.BlockSpec((TSEG_BLK,), lambda i: (i,),
                               memory_space=pltpu.SMEM),
                  pl.BlockSpec((TSEG_BLK,), lambda i: (i,),
                               memory_space=pltpu.SMEM),
                  pl.BlockSpec((TSEG_BLK, D), lambda i: (i, 0))],
        out_specs=[pl.BlockSpec((N, D), lambda i: (0, 0)),
                   pl.BlockSpec((N, D), lambda i: (0, 0))],
        out_shape=[jax.ShapeDtypeStruct((N, D), jnp.float32),
                   jax.ShapeDtypeStruct((N, D), jnp.float32)],
        scratch_shapes=[pltpu.VMEM((N, D), jnp.float32)] * 4,
    )(ridx, cidx, values)


def kernel(values, indices, W, b):
    row_sum, col_sum = _tc_segment_sums(values, indices[0], indices[1])
    a = _matmul_a(values, W[:, :D])
    t, c = _small(row_sum, col_sum, W, b.reshape(1, D))
    return _combine(a, t, indices[0], indices[1], c.reshape(D // 16, 16))


# speedup vs baseline: 2.2280x; 1.0017x over previous
import jax
import jax.numpy as jnp
from jax import lax
from jax.experimental import pallas as pl
from jax.experimental.pallas import tpu as pltpu
from jax.experimental.pallas import tpu_sc as plsc

N = 16384
NNZ = 262144
D = 64
NC, NS = 2, 16
NW = NC * NS


def _mesh():
    return plsc.VectorSubcoreMesh(core_axis_name="c", subcore_axis_name="s",
                                  num_cores=NC, num_subcores=NS)


MM_BLK = 2048


def _mm_body(v_ref, w_ref, o_ref):
    o_ref[...] = lax.dot_general(v_ref[...], w_ref[...],
                                 (((1,), (1,)), ((), ())),
                                 preferred_element_type=jnp.float32)


def _matmul_a(values, w1):
    return pl.pallas_call(
        _mm_body,
        grid=(NNZ // MM_BLK,),
        in_specs=[pl.BlockSpec((MM_BLK, D), lambda i: (i, 0)),
                  pl.BlockSpec((D, D), lambda i: (0, 0))],
        out_specs=pl.BlockSpec((MM_BLK, D), lambda i: (i, 0)),
        out_shape=jax.ShapeDtypeStruct((NNZ, D), jnp.float32),
    )(values, w1)


def _small_body(rsum_ref, csum_ref, w_ref, b_ref, t_ref, c_ref):
    row_sum = rsum_ref[...]
    col_sum = csum_ref[...]
    w = w_ref[...]
    w2 = w[:, D:2 * D]
    w3 = w[:, 2 * D:3 * D]
    w4 = w[:, 3 * D:4 * D]
    dims = (((1,), (1,)), ((), ()))
    trow = lax.dot_general(row_sum, w3, dims, preferred_element_type=jnp.float32)
    tcol = lax.dot_general(col_sum, w2, dims, preferred_element_type=jnp.float32)
    t_ref[...] = jnp.concatenate([trow, tcol], axis=1)
    total = jnp.sum(row_sum, axis=0, keepdims=True) * jnp.float32(1.0 / NNZ)
    c_ref[...] = lax.dot_general(total, w4, dims,
                                 preferred_element_type=jnp.float32) + b_ref[...]


def _small(rsum, csum, w, b2d):
    return pl.pallas_call(
        _small_body,
        out_shape=[jax.ShapeDtypeStruct((N, 2 * D), jnp.float32),
                   jax.ShapeDtypeStruct((1, D), jnp.float32)],
    )(rsum, csum, w, b2d)


C2 = 256
RW = NNZ // NW
C2_ITERS = RW // C2


def _comb_body(a_hbm, t_hbm, rowf_hbm, colf_hbm, c_hbm, out_hbm,
               a_v, gr_v, gc_v, idxr_v, idxc_v, c_v, sem):
    core = lax.axis_index("c")
    sub = lax.axis_index("s")
    wid = sub * NC + core

    pltpu.sync_copy(c_hbm, c_v)
    cs = [c_v[k] for k in range(D // 16)]

    def chunk(g, _):
        base = pl.multiple_of(wid * RW + g * C2, C2)
        pltpu.sync_copy(a_hbm.at[pl.ds(base, C2)], a_v)
        pltpu.sync_copy(rowf_hbm.at[pl.ds(base, C2)], idxr_v)
        pltpu.sync_copy(colf_hbm.at[pl.ds(base, C2)], idxc_v)
        cps = []
        for j in range(C2 // 128):
            cps.append(pltpu.async_copy(t_hbm.at[idxr_v.at[pl.ds(j * 128, 128)]],
                                        gr_v.at[pl.ds(j * 128, 128)], sem))
            cps.append(pltpu.async_copy(t_hbm.at[idxc_v.at[pl.ds(j * 128, 128)]],
                                        gc_v.at[pl.ds(j * 128, 128)], sem))
        for cp in cps:
            cp.wait()

        def rowfn(i, _):
            for k in range(D // 16):
                sl = pl.ds(k * 16, 16)
                t = (a_v[i, sl] + gr_v[i, sl]
                     + gc_v[i, pl.ds(D + k * 16, 16)] + cs[k])
                a_v[i, sl] = jnp.maximum(t, t * jnp.float32(0.01))
            return 0
        lax.fori_loop(0, C2, rowfn, 0)
        pltpu.sync_copy(a_v, out_hbm.at[pl.ds(base, C2)])
        return 0
    lax.fori_loop(0, C2_ITERS, chunk, 0)


def _combine(a, t, rowf, colf, c4):
    f = pl.kernel(
        _comb_body,
        out_type=jax.ShapeDtypeStruct((NNZ, D), jnp.float32),
        mesh=_mesh(),
        scratch_types=[
            pltpu.VMEM((C2, D), jnp.float32),
            pltpu.VMEM((C2, 2 * D), jnp.float32),
            pltpu.VMEM((C2, 2 * D), jnp.float32),
            pltpu.VMEM((C2,), jnp.int32),
            pltpu.VMEM((C2,), jnp.int32),
            pltpu.VMEM((D // 16, 16), jnp.float32),
            pltpu.SemaphoreType.DMA,
        ],
    )
    return f(a, t, rowf, colf, c4)


TSEG_BLK = 4096


def _tcseg_body(ridx_ref, cidx_ref, vals_ref, outr_ref, outc_ref,
                ar0, ar1, ac0, ac1):
    g = pl.program_id(0)
    accs = (ar0, ar1, ac0, ac1)

    @pl.when(g == 0)
    def _():
        for a in accs:
            a[...] = jnp.zeros_like(a)

    def body(i, _):
        for k in range(2):
            v = vals_ref[pl.ds(i * 2 + k, 1), :]
            rr = ridx_ref[i * 2 + k]
            cc = cidx_ref[i * 2 + k]
            accs[k][pl.ds(rr, 1), :] += v
            accs[2 + k][pl.ds(cc, 1), :] += v
        return 0
    lax.fori_loop(0, TSEG_BLK // 2, body, 0)

    @pl.when(g == (NNZ // TSEG_BLK) - 1)
    def _():
        outr_ref[...] = ar0[...] + ar1[...]
        outc_ref[...] = ac0[...] + ac1[...]


def _tc_segment_sums(values, ridx, cidx):
    return pl.pallas_call(
        _tcseg_body,
        grid=(NNZ // TSEG_BLK,),
        in_specs=[pl.BlockSpec((TSEG_BLK,), lambda i: (i,),
                               memory_space=pltpu.SMEM),
                  pl.BlockSpec((TSEG_BLK,), lambda i: (i,),
                               memory_space=pltpu.SMEM),
                  pl.BlockSpec((TSEG_BLK, D), lambda i: (i, 0))],
        out_specs=[pl.BlockSpec((N, D), lambda i: (0, 0)),
                   pl.BlockSpec((N, D), lambda i: (0, 0))],
        out_shape=[jax.ShapeDtypeStruct((N, D), jnp.float32),
                   jax.ShapeDtypeStruct((N, D), jnp.float32)],
        scratch_shapes=[pltpu.VMEM((N, D), jnp.float32)] * 4,
    )(ridx, cidx, values)


def kernel(values, indices, W, b):
    row_sum, col_sum = _tc_segment_sums(values, indices[0], indices[1])
    a = _matmul_a(values, W[:, :D])
    t, c = _small(row_sum, col_sum, W, b.reshape(1, D))
    return _combine(a, t, indices[0], indices[1], c.reshape(D // 16, 16))
